# Initial kernel scaffold; baseline (speedup 1.0000x reference)
import jax
import jax.numpy as jnp
from jax.experimental import pallas as pl


def kernel(x, last_update, edge_index, t, msg, W1, b1, W2, b2):
    raise NotImplementedError("write your pallas kernel here")


# speedup vs baseline: 12.5706x; 12.5706x over previous
import functools

import jax
import jax.numpy as jnp
from jax import lax
from jax.experimental import pallas as pl
from jax.experimental.pallas import tpu as pltpu
from jax.experimental.pallas import tpu_sc as plsc

N = 10000
E = 320000
D = 128

NC = 2
NS = 16
NW = NC * NS
EPW = E // NW
C = 80
NCHUNK = EPW // C
NPAD = 10240
RPT = NPAD // NS

_MESH = plsc.VectorSubcoreMesh(
    core_axis_name="c", subcore_axis_name="s", num_cores=NC, num_subcores=NS
)


@functools.partial(
    pl.kernel,
    out_type=jax.ShapeDtypeStruct((NC, NPAD, 16), jnp.float32),
    mesh=_MESH,
    scratch_types=[
        pltpu.VMEM((C,), jnp.int32),
        pltpu.VMEM((C, 16), jnp.float32),
        pltpu.VMEM_SHARED((NPAD, 16), jnp.float32),
    ],
)
def _deg_pass(dst_hbm, ones_hbm, zeros_hbm, out_hbm, didx_v, ones_v, acc_sh):
    cid = lax.axis_index("c")
    sid = lax.axis_index("s")
    w = sid * NC + cid
    r0 = sid * RPT
    pltpu.sync_copy(zeros_hbm, acc_sh.at[pl.ds(r0, RPT)])
    pltpu.sync_copy(ones_hbm, ones_v)
    plsc.subcore_barrier()

    def body(j, carry):
        base = w * EPW + j * C
        pltpu.sync_copy(dst_hbm.at[pl.ds(base, C)], didx_v)
        pltpu.sync_copy(ones_v, acc_sh.at[didx_v], add=True)
        return carry

    lax.fori_loop(0, NCHUNK, body, 0)
    plsc.subcore_barrier()
    pltpu.sync_copy(acc_sh.at[pl.ds(r0, RPT)], out_hbm.at[cid, pl.ds(r0, RPT)])


@functools.partial(
    pl.kernel,
    out_type=jax.ShapeDtypeStruct((NC, NPAD, D), jnp.float32),
    mesh=_MESH,
    scratch_types=[
        pltpu.VMEM((C,), jnp.int32),
        pltpu.VMEM((C,), jnp.int32),
        pltpu.VMEM((C, D), jnp.float32),
        pltpu.VMEM_SHARED((NPAD, D), jnp.float32),
        pltpu.SemaphoreType.DMA,
    ],
)
def _edge_pass(y_hbm, src_hbm, dst_hbm, zeros_hbm, out_hbm,
               sidx_v, didx_v, rows_v, acc_sh, sem):
    cid = lax.axis_index("c")
    sid = lax.axis_index("s")
    w = sid * NC + cid
    r0 = sid * RPT
    pltpu.sync_copy(zeros_hbm, acc_sh.at[pl.ds(r0, RPT)])
    plsc.subcore_barrier()

    def body(j, carry):
        base = w * EPW + j * C
        pltpu.sync_copy(src_hbm.at[pl.ds(base, C)], sidx_v)
        pltpu.sync_copy(dst_hbm.at[pl.ds(base, C)], didx_v)
        pltpu.async_copy(y_hbm.at[sidx_v], rows_v, sem).wait()
        pltpu.sync_copy(rows_v, acc_sh.at[didx_v], add=True)
        return carry

    lax.fori_loop(0, NCHUNK, body, 0)
    plsc.subcore_barrier()
    pltpu.sync_copy(acc_sh.at[pl.ds(r0, RPT)], out_hbm.at[cid, pl.ds(r0, RPT)])


B = 2000


def _dinv(dp0_ref, dp1_ref):
    deg = dp0_ref[:, 0:1] + dp1_ref[:, 0:1] + 1.0
    return lax.rsqrt(deg)


def _tc1_body(x_ref, w1_ref, dp0_ref, dp1_ref, y1_ref):
    y1_ref[...] = jnp.dot(
        x_ref[...], w1_ref[...], preferred_element_type=jnp.float32
    ) * _dinv(dp0_ref, dp1_ref)


def _tc2_body(a0_ref, a1_ref, y1_ref, dp0_ref, dp1_ref, b1_ref, w2_ref, y2_ref):
    dinv = _dinv(dp0_ref, dp1_ref)
    h = dinv * (a0_ref[...] + a1_ref[...] + y1_ref[...]) + b1_ref[...]
    h = jnp.maximum(h, 0.0)
    y2_ref[...] = jnp.dot(h, w2_ref[...], preferred_element_type=jnp.float32) * dinv


def _tc3_body(a0_ref, a1_ref, y2_ref, dp0_ref, dp1_ref, b2_ref, out_ref):
    dinv = _dinv(dp0_ref, dp1_ref)
    out_ref[...] = dinv * (a0_ref[...] + a1_ref[...] + y2_ref[...]) + b2_ref[...]


_row_blk = pl.BlockSpec((B, D), lambda i: (i, 0))
_deg_blk = pl.BlockSpec((B, 16), lambda i: (i, 0))
_w_blk = pl.BlockSpec((D, D), lambda i: (0, 0))
_b_blk = pl.BlockSpec((1, D), lambda i: (0, 0))
_grid = (N // B,)
_out_nd = jax.ShapeDtypeStruct((N, D), jnp.float32)

_tc1 = pl.pallas_call(
    _tc1_body, grid=_grid,
    in_specs=[_row_blk, _w_blk, _deg_blk, _deg_blk],
    out_specs=_row_blk, out_shape=_out_nd,
)
_tc2 = pl.pallas_call(
    _tc2_body, grid=_grid,
    in_specs=[_row_blk, _row_blk, _row_blk, _deg_blk, _deg_blk, _b_blk, _w_blk],
    out_specs=_row_blk, out_shape=_out_nd,
)
_tc3 = pl.pallas_call(
    _tc3_body, grid=_grid,
    in_specs=[_row_blk, _row_blk, _row_blk, _deg_blk, _deg_blk, _b_blk],
    out_specs=_row_blk, out_shape=_out_nd,
)


def kernel(x, last_update, edge_index, t, msg, W1, b1, W2, b2):
    del last_update, t, msg
    src = edge_index[0].astype(jnp.int32)
    dst = edge_index[1].astype(jnp.int32)
    ones16 = jnp.ones((C, 16), jnp.float32)
    z16 = jnp.zeros((RPT, 16), jnp.float32)
    zD = jnp.zeros((RPT, D), jnp.float32)

    degp = _deg_pass(dst, ones16, z16)
    dp0, dp1 = degp[0], degp[1]

    y1 = _tc1(x, W1, dp0, dp1)
    a1 = _edge_pass(y1, src, dst, zD)
    y2 = _tc2(a1[0], a1[1], y1, dp0, dp1, b1.reshape(1, D), W2)
    a2 = _edge_pass(y2, src, dst, zD)
    out = _tc3(a2[0], a2[1], y2, dp0, dp1, b2.reshape(1, D))
    return out


# speedup vs baseline: 19.6366x; 1.5621x over previous
import functools

import jax
import jax.numpy as jnp
from jax import lax
from jax.experimental import pallas as pl
from jax.experimental.pallas import tpu as pltpu
from jax.experimental.pallas import tpu_sc as plsc

N = 10000
E = 320000
D = 128

NC = 2
NS = 16
NW = NC * NS
EPW = E // NW
C = 40
NCHUNK = EPW // C
KD = 10
NOUTD = NCHUNK // KD
K = 5
NOUT = NCHUNK // K
NPAD = 10240
RPT = NPAD // NS

_MESH = plsc.VectorSubcoreMesh(
    core_axis_name="c", subcore_axis_name="s", num_cores=NC, num_subcores=NS
)


def _worker(pfx=None):
    cid = lax.axis_index("c")
    sid = lax.axis_index("s")
    return sid * NC + cid, cid, sid


@functools.partial(
    pl.kernel,
    out_type=jax.ShapeDtypeStruct((NC, NPAD, 16), jnp.float32),
    mesh=_MESH,
    scratch_types=(
        [pltpu.VMEM((C, 16), jnp.float32)]
        + [pltpu.VMEM((C,), jnp.int32) for _ in range(KD)]
        + [pltpu.VMEM_SHARED((NPAD, 16), jnp.float32)]
        + [pltpu.SemaphoreType.DMA for _ in range(2 * KD)]
    ),
)
def _deg_pass(dst_hbm, ones_hbm, zeros_hbm, out_hbm, *scr):
    ones_v = scr[0]
    didx = scr[1:1 + KD]
    acc_sh = scr[1 + KD]
    dsem = scr[2 + KD:2 + 2 * KD]
    ssem = scr[2 + 2 * KD:2 + 3 * KD]
    w, cid, sid = _worker()
    r0 = sid * RPT
    pltpu.sync_copy(zeros_hbm, acc_sh.at[pl.ds(r0, RPT)])
    pltpu.sync_copy(ones_hbm, ones_v)
    plsc.subcore_barrier()

    def outer(j0, carry):
        dd = []
        for i in range(KD):
            base = w * EPW + (j0 * KD + i) * C
            dd.append(pltpu.async_copy(dst_hbm.at[pl.ds(base, C)], didx[i], dsem[i]))
        sd = []
        for i in range(KD):
            dd[i].wait()
            sd.append(pltpu.async_copy(ones_v, acc_sh.at[didx[i]], ssem[i], add=True))
        for i in range(KD):
            sd[i].wait()
        return carry

    lax.fori_loop(0, NOUTD, outer, 0)
    plsc.subcore_barrier()
    pltpu.sync_copy(acc_sh.at[pl.ds(r0, RPT)], out_hbm.at[cid, pl.ds(r0, RPT)])


@functools.partial(
    pl.kernel,
    out_type=jax.ShapeDtypeStruct((NC, NPAD, D), jnp.float32),
    mesh=_MESH,
    scratch_types=(
        [pltpu.VMEM((C, D), jnp.float32) for _ in range(K)]
        + [pltpu.VMEM((C,), jnp.int32) for _ in range(K)]
        + [pltpu.VMEM((C,), jnp.int32) for _ in range(K)]
        + [pltpu.VMEM_SHARED((NPAD, D), jnp.float32)]
        + [pltpu.SemaphoreType.DMA for _ in range(4 * K)]
    ),
)
def _edge_pass(y_hbm, src_hbm, dst_hbm, zeros_hbm, out_hbm, *scr):
    rows = scr[0:K]
    sidx = scr[K:2 * K]
    didx = scr[2 * K:3 * K]
    acc_sh = scr[3 * K]
    isem = scr[3 * K + 1:4 * K + 1]
    gsem = scr[4 * K + 1:5 * K + 1]
    dsem = scr[5 * K + 1:6 * K + 1]
    ssem = scr[6 * K + 1:7 * K + 1]
    w, cid, sid = _worker()
    r0 = sid * RPT
    pltpu.sync_copy(zeros_hbm, acc_sh.at[pl.ds(r0, RPT)])
    plsc.subcore_barrier()

    def outer(j0, carry):
        id_, dd = [], []
        for i in range(K):
            base = w * EPW + (j0 * K + i) * C
            id_.append(pltpu.async_copy(src_hbm.at[pl.ds(base, C)], sidx[i], isem[i]))
            dd.append(pltpu.async_copy(dst_hbm.at[pl.ds(base, C)], didx[i], dsem[i]))
        gd = []
        for i in range(K):
            id_[i].wait()
            gd.append(pltpu.async_copy(y_hbm.at[sidx[i]], rows[i], gsem[i]))
        for i in range(K):
            gd[i].wait()
        for i in range(K):
            dd[i].wait()
            pltpu.sync_copy(rows[i], acc_sh.at[didx[i]], add=True)
        return carry

    lax.fori_loop(0, NOUT, outer, 0)
    plsc.subcore_barrier()
    pltpu.sync_copy(acc_sh.at[pl.ds(r0, RPT)], out_hbm.at[cid, pl.ds(r0, RPT)])


B = 2000


def _dinv(dp0_ref, dp1_ref):
    deg = dp0_ref[:, 0:1] + dp1_ref[:, 0:1] + 1.0
    return lax.rsqrt(deg)


def _tc1_body(x_ref, w1_ref, dp0_ref, dp1_ref, y1_ref):
    y1_ref[...] = jnp.dot(
        x_ref[...], w1_ref[...], preferred_element_type=jnp.float32
    ) * _dinv(dp0_ref, dp1_ref)


def _tc2_body(a0_ref, a1_ref, y1_ref, dp0_ref, dp1_ref, b1_ref, w2_ref, y2_ref):
    dinv = _dinv(dp0_ref, dp1_ref)
    h = dinv * (a0_ref[...] + a1_ref[...] + y1_ref[...]) + b1_ref[...]
    h = jnp.maximum(h, 0.0)
    y2_ref[...] = jnp.dot(h, w2_ref[...], preferred_element_type=jnp.float32) * dinv


def _tc3_body(a0_ref, a1_ref, y2_ref, dp0_ref, dp1_ref, b2_ref, out_ref):
    dinv = _dinv(dp0_ref, dp1_ref)
    out_ref[...] = dinv * (a0_ref[...] + a1_ref[...] + y2_ref[...]) + b2_ref[...]


_row_blk = pl.BlockSpec((B, D), lambda i: (i, 0))
_deg_blk = pl.BlockSpec((B, 16), lambda i: (i, 0))
_w_blk = pl.BlockSpec((D, D), lambda i: (0, 0))
_b_blk = pl.BlockSpec((1, D), lambda i: (0, 0))
_grid = (N // B,)
_out_nd = jax.ShapeDtypeStruct((N, D), jnp.float32)

_tc1 = pl.pallas_call(
    _tc1_body, grid=_grid,
    in_specs=[_row_blk, _w_blk, _deg_blk, _deg_blk],
    out_specs=_row_blk, out_shape=_out_nd,
)
_tc2 = pl.pallas_call(
    _tc2_body, grid=_grid,
    in_specs=[_row_blk, _row_blk, _row_blk, _deg_blk, _deg_blk, _b_blk, _w_blk],
    out_specs=_row_blk, out_shape=_out_nd,
)
_tc3 = pl.pallas_call(
    _tc3_body, grid=_grid,
    in_specs=[_row_blk, _row_blk, _row_blk, _deg_blk, _deg_blk, _b_blk],
    out_specs=_row_blk, out_shape=_out_nd,
)


def kernel(x, last_update, edge_index, t, msg, W1, b1, W2, b2):
    del last_update, t, msg
    src = edge_index[0].astype(jnp.int32)
    dst = edge_index[1].astype(jnp.int32)
    ones16 = jnp.ones((C, 16), jnp.float32)
    z16 = jnp.zeros((RPT, 16), jnp.float32)
    zD = jnp.zeros((RPT, D), jnp.float32)

    degp = _deg_pass(dst, ones16, z16)
    dp0, dp1 = degp[0], degp[1]

    y1 = _tc1(x, W1, dp0, dp1)
    a1 = _edge_pass(y1, src, dst, zD)
    y2 = _tc2(a1[0], a1[1], y1, dp0, dp1, b1.reshape(1, D), W2)
    a2 = _edge_pass(y2, src, dst, zD)
    out = _tc3(a2[0], a2[1], y2, dp0, dp1, b2.reshape(1, D))
    return out


# speedup vs baseline: 20.6803x; 1.0532x over previous
import functools

import jax
import jax.numpy as jnp
from jax import lax
from jax.experimental import pallas as pl
from jax.experimental.pallas import tpu as pltpu
from jax.experimental.pallas import tpu_sc as plsc

N = 10000
E = 320000
D = 128

NC = 2
NS = 16
NW = NC * NS
EPW = E // NW
C = 40
NCHUNK = EPW // C
KD = 10
NOUTD = NCHUNK // KD
K = 5
NOUT = NCHUNK // K
NPAD = 10240
RPT = NPAD // NS

_MESH = plsc.VectorSubcoreMesh(
    core_axis_name="c", subcore_axis_name="s", num_cores=NC, num_subcores=NS
)


def _worker(pfx=None):
    cid = lax.axis_index("c")
    sid = lax.axis_index("s")
    return sid * NC + cid, cid, sid


@functools.partial(
    pl.kernel,
    out_type=jax.ShapeDtypeStruct((NC, NPAD, 16), jnp.float32),
    mesh=_MESH,
    scratch_types=(
        [pltpu.VMEM((C, 16), jnp.float32)]
        + [pltpu.VMEM((C,), jnp.int32) for _ in range(KD)]
        + [pltpu.VMEM_SHARED((NPAD, 16), jnp.float32)]
        + [pltpu.SemaphoreType.DMA for _ in range(2 * KD)]
    ),
)
def _deg_pass(dst_hbm, ones_hbm, zeros_hbm, out_hbm, *scr):
    ones_v = scr[0]
    didx = scr[1:1 + KD]
    acc_sh = scr[1 + KD]
    dsem = scr[2 + KD:2 + 2 * KD]
    ssem = scr[2 + 2 * KD:2 + 3 * KD]
    w, cid, sid = _worker()
    r0 = sid * RPT
    pltpu.sync_copy(zeros_hbm, acc_sh.at[pl.ds(r0, RPT)])
    pltpu.sync_copy(ones_hbm, ones_v)
    plsc.subcore_barrier()

    def outer(j0, carry):
        dd = []
        for i in range(KD):
            base = w * EPW + (j0 * KD + i) * C
            dd.append(pltpu.async_copy(dst_hbm.at[pl.ds(base, C)], didx[i], dsem[i]))
        sd = []
        for i in range(KD):
            dd[i].wait()
            sd.append(pltpu.async_copy(ones_v, acc_sh.at[didx[i]], ssem[i], add=True))
        for i in range(KD):
            sd[i].wait()
        return carry

    lax.fori_loop(0, NOUTD, outer, 0)
    plsc.subcore_barrier()
    pltpu.sync_copy(acc_sh.at[pl.ds(r0, RPT)], out_hbm.at[cid, pl.ds(r0, RPT)])


@functools.partial(
    pl.kernel,
    out_type=jax.ShapeDtypeStruct((NC, NPAD, D), jnp.float32),
    mesh=_MESH,
    scratch_types=(
        [pltpu.VMEM((C, D), jnp.float32) for _ in range(K)]
        + [pltpu.VMEM((C,), jnp.int32) for _ in range(K)]
        + [pltpu.VMEM((C,), jnp.int32) for _ in range(K)]
        + [pltpu.VMEM_SHARED((NPAD, D), jnp.float32)]
        + [pltpu.SemaphoreType.DMA for _ in range(4 * K)]
    ),
)
def _edge_pass(y_hbm, src_hbm, dst_hbm, zeros_hbm, out_hbm, *scr):
    rows = scr[0:K]
    sidx = scr[K:2 * K]
    didx = scr[2 * K:3 * K]
    acc_sh = scr[3 * K]
    isem = scr[3 * K + 1:4 * K + 1]
    gsem = scr[4 * K + 1:5 * K + 1]
    dsem = scr[5 * K + 1:6 * K + 1]
    ssem = scr[6 * K + 1:7 * K + 1]
    w, cid, sid = _worker()
    r0 = sid * RPT
    pltpu.sync_copy(zeros_hbm, acc_sh.at[pl.ds(r0, RPT)])
    plsc.subcore_barrier()

    def outer(j0, carry):
        id_, dd = [], []
        for i in range(K):
            base = w * EPW + (j0 * K + i) * C
            id_.append(pltpu.async_copy(src_hbm.at[pl.ds(base, C)], sidx[i], isem[i]))
            dd.append(pltpu.async_copy(dst_hbm.at[pl.ds(base, C)], didx[i], dsem[i]))
        gd = []
        for i in range(K):
            id_[i].wait()
            gd.append(pltpu.async_copy(y_hbm.at[sidx[i]], rows[i], gsem[i]))
        for i in range(K):
            gd[i].wait()
            dd[i].wait()
            pltpu.sync_copy(rows[i], acc_sh.at[didx[i]], add=True)
        return carry

    lax.fori_loop(0, NOUT, outer, 0)
    plsc.subcore_barrier()
    pltpu.sync_copy(acc_sh.at[pl.ds(r0, RPT)], out_hbm.at[cid, pl.ds(r0, RPT)])


B = 2000


def _dinv(dp0_ref, dp1_ref):
    deg = dp0_ref[:, 0:1] + dp1_ref[:, 0:1] + 1.0
    return lax.rsqrt(deg)


def _tc1_body(x_ref, w1_ref, dp0_ref, dp1_ref, y1_ref):
    y1_ref[...] = jnp.dot(
        x_ref[...], w1_ref[...], preferred_element_type=jnp.float32
    ) * _dinv(dp0_ref, dp1_ref)


def _tc2_body(a0_ref, a1_ref, y1_ref, dp0_ref, dp1_ref, b1_ref, w2_ref, y2_ref):
    dinv = _dinv(dp0_ref, dp1_ref)
    h = dinv * (a0_ref[...] + a1_ref[...] + y1_ref[...]) + b1_ref[...]
    h = jnp.maximum(h, 0.0)
    y2_ref[...] = jnp.dot(h, w2_ref[...], preferred_element_type=jnp.float32) * dinv


def _tc3_body(a0_ref, a1_ref, y2_ref, dp0_ref, dp1_ref, b2_ref, out_ref):
    dinv = _dinv(dp0_ref, dp1_ref)
    out_ref[...] = dinv * (a0_ref[...] + a1_ref[...] + y2_ref[...]) + b2_ref[...]


_row_blk = pl.BlockSpec((B, D), lambda i: (i, 0))
_deg_blk = pl.BlockSpec((B, 16), lambda i: (i, 0))
_w_blk = pl.BlockSpec((D, D), lambda i: (0, 0))
_b_blk = pl.BlockSpec((1, D), lambda i: (0, 0))
_grid = (N // B,)
_out_nd = jax.ShapeDtypeStruct((N, D), jnp.float32)

_tc1 = pl.pallas_call(
    _tc1_body, grid=_grid,
    in_specs=[_row_blk, _w_blk, _deg_blk, _deg_blk],
    out_specs=_row_blk, out_shape=_out_nd,
)
_tc2 = pl.pallas_call(
    _tc2_body, grid=_grid,
    in_specs=[_row_blk, _row_blk, _row_blk, _deg_blk, _deg_blk, _b_blk, _w_blk],
    out_specs=_row_blk, out_shape=_out_nd,
)
_tc3 = pl.pallas_call(
    _tc3_body, grid=_grid,
    in_specs=[_row_blk, _row_blk, _row_blk, _deg_blk, _deg_blk, _b_blk],
    out_specs=_row_blk, out_shape=_out_nd,
)


def kernel(x, last_update, edge_index, t, msg, W1, b1, W2, b2):
    del last_update, t, msg
    src = edge_index[0].astype(jnp.int32)
    dst = edge_index[1].astype(jnp.int32)
    ones16 = jnp.ones((C, 16), jnp.float32)
    z16 = jnp.zeros((RPT, 16), jnp.float32)
    zD = jnp.zeros((RPT, D), jnp.float32)

    degp = _deg_pass(dst, ones16, z16)
    dp0, dp1 = degp[0], degp[1]

    y1 = _tc1(x, W1, dp0, dp1)
    a1 = _edge_pass(y1, src, dst, zD)
    y2 = _tc2(a1[0], a1[1], y1, dp0, dp1, b1.reshape(1, D), W2)
    a2 = _edge_pass(y2, src, dst, zD)
    out = _tc3(a2[0], a2[1], y2, dp0, dp1, b2.reshape(1, D))
    return out
